# baseline (speedup 1.0000x reference)
import functools

import jax
import jax.numpy as jnp
from jax.experimental import pallas as pl
from jax.experimental.pallas import tpu as pltpu
from jax.experimental.pallas import tpu_sc as plsc

_NUM_CHUNKS = 4
_GATHER_WINDOW = 1600
_BLOCK_M = 1024


def _sc_gather(table, idx_flat, num_idx, embed_dim):
    mesh = plsc.VectorSubcoreMesh(core_axis_name="c", subcore_axis_name="s")
    gw = _GATHER_WINDOW

    @functools.partial(
        pl.kernel,
        mesh=mesh,
        out_type=jax.ShapeDtypeStruct((num_idx, embed_dim), jnp.float32),
        compiler_params=pltpu.CompilerParams(use_tc_tiling_on_sc=False),
    )
    def gather_kernel(table_hbm, i_hbm, o_hbm):
        def body(i_vmem, o_vmem):
            pltpu.sync_copy(table_hbm.at[i_vmem.at[0]], o_vmem)

        pltpu.emit_pipeline(
            body,
            grid=(num_idx // gw,),
            in_specs=[pl.BlockSpec((1, gw), lambda i: (0, i))],
            out_specs=[pl.BlockSpec((gw, embed_dim), lambda i: (i, 0))],
            core_axis_name=("c", "s"),
            dimension_semantics=(pltpu.PARALLEL,),
        )(i_hbm, o_hbm)

    return gather_kernel(table, idx_flat)


def _tc_matmul(flat, W_bf16, b, block_m=_BLOCK_M):
    B, K = flat.shape
    _, N = W_bf16.shape

    def mm_body(x_ref, w_ref, b_ref, o_ref):
        o_ref[...] = (
            jnp.dot(x_ref[...], w_ref[...], preferred_element_type=jnp.float32)
            + b_ref[...]
        )

    return pl.pallas_call(
        mm_body,
        grid=(B // block_m,),
        in_specs=[
            pl.BlockSpec((block_m, K), lambda i: (i, 0)),
            pl.BlockSpec((K, N), lambda i: (0, 0)),
            pl.BlockSpec((1, N), lambda i: (0, 0)),
        ],
        out_specs=pl.BlockSpec((block_m, N), lambda i: (i, 0)),
        out_shape=jax.ShapeDtypeStruct((B, N), jnp.float32),
    )(flat, W_bf16, b.reshape(1, N))


def kernel(obs, table, W, b):
    B, OD = obs.shape
    V, E = table.shape
    K, N = W.shape

    W_bf16 = W.astype(jnp.bfloat16)
    cb = B // _NUM_CHUNKS

    rows_chunks = []
    for c in range(_NUM_CHUNKS):
        obs_c = obs[c * cb : (c + 1) * cb]
        idx_c = obs_c.reshape(1, cb * OD)
        rows_chunks.append(_sc_gather(table, idx_c, cb * OD, E))
    outs = [
        _tc_matmul(rows.astype(jnp.bfloat16).reshape(cb, OD * E), W_bf16, b)
        for rows in rows_chunks
    ]
    return jnp.concatenate(outs, axis=0)


# speedup vs baseline: 3.4795x; 3.4795x over previous
import functools

import jax
import jax.numpy as jnp
from jax.experimental import pallas as pl
from jax.experimental.pallas import tpu as pltpu
from jax.experimental.pallas import tpu_sc as plsc

_NUM_CHUNKS = 4
_GATHER_WINDOW = 3200
_BLOCK_M = 1024


def _sc_gather(table, idx_flat, num_idx, embed_dim):
    mesh = plsc.VectorSubcoreMesh(core_axis_name="c", subcore_axis_name="s")
    gw = _GATHER_WINDOW

    @functools.partial(
        pl.kernel,
        mesh=mesh,
        out_type=jax.ShapeDtypeStruct((num_idx, embed_dim), jnp.float32),
        compiler_params=pltpu.CompilerParams(use_tc_tiling_on_sc=False),
    )
    def gather_kernel(table_hbm, i_hbm, o_hbm):
        def body(i_vmem, o_vmem):
            pltpu.sync_copy(table_hbm.at[i_vmem.at[0]], o_vmem)

        pltpu.emit_pipeline(
            body,
            grid=(num_idx // gw,),
            in_specs=[pl.BlockSpec((1, gw), lambda i: (0, i))],
            out_specs=[pl.BlockSpec((gw, embed_dim), lambda i: (i, 0))],
            core_axis_name=("c", "s"),
            dimension_semantics=(pltpu.PARALLEL,),
        )(i_hbm, o_hbm)

    return gather_kernel(table, idx_flat)


def _tc_matmul(flat, W_bf16, b, block_m=_BLOCK_M):
    B, K = flat.shape
    _, N = W_bf16.shape

    def mm_body(x_ref, w_ref, b_ref, o_ref):
        o_ref[...] = (
            jnp.dot(x_ref[...].astype(jnp.bfloat16), w_ref[...], preferred_element_type=jnp.float32)
            + b_ref[...]
        )

    return pl.pallas_call(
        mm_body,
        grid=(B // block_m,),
        in_specs=[
            pl.BlockSpec((block_m, K), lambda i: (i, 0)),
            pl.BlockSpec((K, N), lambda i: (0, 0)),
            pl.BlockSpec((1, N), lambda i: (0, 0)),
        ],
        out_specs=pl.BlockSpec((block_m, N), lambda i: (i, 0)),
        out_shape=jax.ShapeDtypeStruct((B, N), jnp.float32),
    )(flat, W_bf16, b.reshape(1, N))


def kernel(obs, table, W, b):
    B, OD = obs.shape
    V, E = table.shape
    K, N = W.shape

    W_bf16 = W.astype(jnp.bfloat16)
    cb = B // _NUM_CHUNKS

    rows_chunks = []
    for c in range(_NUM_CHUNKS):
        obs_c = obs[c * cb : (c + 1) * cb]
        idx_c = obs_c.reshape(1, cb * OD)
        rows_chunks.append(_sc_gather(table, idx_c, cb * OD, E))
    outs = [
        _tc_matmul(rows.reshape(cb, OD * E), W_bf16, b)
        for rows in rows_chunks
    ]
    return jnp.concatenate(outs, axis=0)
